# Initial kernel scaffold; baseline (speedup 1.0000x reference)
import functools

import jax
import jax.numpy as jnp
from jax import lax
from jax.experimental import pallas as pl
from jax.experimental.pallas import tpu as pltpu
from jax.experimental.pallas import tpu_sc as plsc

_VOCAB = 1000000
_DIM = 64
_BATCH = 16384
_HIST = 50
_N = _BATCH * _HIST
_NW = 32
_PER_W = _N // _NW
_CHUNK = 1024
_NCHUNK = _PER_W // _CHUNK
_SUB = 128
_NSUB = _CHUNK // _SUB
_GROUPS = _CHUNK // 16


def _dequant_group(rows_v, scl_v, out_v, g):
    row_ids = g * 16 + lax.iota(jnp.int32, 16)
    sw = scl_v[pl.ds(g * 16, 16)]
    s_f16 = plsc.bitcast(sw, jnp.float16)
    s0, s1 = plsc.unpack(s_f16, format=plsc.PackFormat.INTERLEAVED,
                         preferred_element_type=jnp.float32)
    for w in range(16):
        col = jnp.full((16,), w, jnp.int32)
        v = plsc.load_gather(rows_v, [row_ids, col])
        s = s0 if w < 8 else s1
        b0 = ((v << 24) >> 24).astype(jnp.float32) * s
        b1 = ((v << 16) >> 24).astype(jnp.float32) * s
        b2 = ((v << 8) >> 24).astype(jnp.float32) * s
        b3 = (v >> 24).astype(jnp.float32) * s
        w01 = plsc.bitcast(
            plsc.pack(b0, b1, format=plsc.PackFormat.INTERLEAVED,
                      preferred_element_type=jnp.float16), jnp.int32)
        w23 = plsc.bitcast(
            plsc.pack(b2, b3, format=plsc.PackFormat.INTERLEAVED,
                      preferred_element_type=jnp.float16), jnp.int32)
        plsc.store_scatter(out_v, [row_ids, jnp.full((16,), 2 * w, jnp.int32)], w01)
        plsc.store_scatter(out_v, [row_ids, jnp.full((16,), 2 * w + 1, jnp.int32)], w23)


def _sc_body(idx_hbm, w_hbm, s_hbm, out_hbm, idx_v, scl_v, rows_v, out_v,
             sem_w, sem_s):
    wid = lax.axis_index("s") * 2 + lax.axis_index("c")
    base_row = wid * _PER_W

    def chunk_body(c, _):
        r0 = base_row + c * _CHUNK
        pltpu.sync_copy(idx_hbm.at[pl.ds(r0 // _SUB, _NSUB)], idx_v)
        copies = []
        for j in range(_NSUB):
            copies.append(pltpu.async_copy(
                w_hbm.at[idx_v.at[j]], rows_v.at[pl.ds(j * _SUB, _SUB)], sem_w))
            copies.append(pltpu.async_copy(
                s_hbm.at[idx_v.at[j]], scl_v.at[pl.ds(j * _SUB, _SUB)], sem_s))
        for cp in copies:
            cp.wait()

        def group_body(g, _):
            _dequant_group(rows_v, scl_v, out_v, g)
            return 0

        lax.fori_loop(0, _GROUPS, group_body, 0)
        pltpu.sync_copy(out_v, out_hbm.at[pl.ds(r0, _CHUNK)])
        return 0

    lax.fori_loop(0, _NCHUNK, chunk_body, 0)


@functools.partial(
    pl.kernel,
    out_type=jax.ShapeDtypeStruct((_N, 16 * 2), jnp.int32),
    mesh=plsc.VectorSubcoreMesh(core_axis_name="c", subcore_axis_name="s"),
    scratch_types=[
        pltpu.VMEM((_NSUB, _SUB), jnp.int32),
        pltpu.VMEM((_CHUNK,), jnp.int32),
        pltpu.VMEM((_CHUNK, 16), jnp.int32),
        pltpu.VMEM((_CHUNK, 32), jnp.int32),
        pltpu.SemaphoreType.DMA,
        pltpu.SemaphoreType.DMA,
    ],
)
def _sc_lookup(idx_hbm, w_hbm, s_hbm, out_hbm, idx_v, scl_v, rows_v, out_v,
               sem_w, sem_s):
    _sc_body(idx_hbm, w_hbm, s_hbm, out_hbm, idx_v, scl_v, rows_v, out_v,
             sem_w, sem_s)


def kernel(indices, weight_int8, scales_fp16):
    V, D = weight_int8.shape
    idx2d = indices.reshape(_N // _SUB, _SUB)
    w_i32 = lax.bitcast_convert_type(weight_int8.reshape(V, D // 4, 4), jnp.int32)
    s_i32 = lax.bitcast_convert_type(scales_fp16, jnp.int32)
    out_i32 = _sc_lookup(idx2d, w_i32, s_i32)
    out = lax.bitcast_convert_type(out_i32, jnp.float16)
    return out.reshape(indices.shape + (D,))


# speedup vs baseline: 2.1902x; 2.1902x over previous
import functools

import jax
import jax.numpy as jnp
from jax import lax
from jax.experimental import pallas as pl
from jax.experimental.pallas import tpu as pltpu
from jax.experimental.pallas import tpu_sc as plsc

_VOCAB = 1000000
_DIM = 64
_BATCH = 16384
_HIST = 50
_N = _BATCH * _HIST
_NW = 32
_PER_W = _N // _NW
_CHUNK = 1024
_NCHUNK = _PER_W // _CHUNK
_SUB = 128
_NSUB = _CHUNK // _SUB
_GROUPS = _CHUNK // 16


def _dequant_group(rows_v, scl_v, out_v, g):
    row_ids = g * 16 + lax.iota(jnp.int32, 16)
    sw = scl_v[pl.ds(g * 16, 16)]
    sdup0 = plsc.bitcast((sw & 0xFFFF) | (sw << 16), jnp.float16)
    sdup1 = plsc.bitcast(lax.shift_right_logical(sw, 16)
                         | (sw & jnp.int32(-65536)), jnp.float16)
    c128 = jnp.full((32,), 128.0, jnp.float16)
    for w in range(16):
        col = jnp.full((16,), w, jnp.int32)
        v = plsc.load_gather(rows_v, [row_ids, col])
        vx = v ^ jnp.int32(-2139062144)
        p01 = (vx & 0xFF) | ((vx & 0xFF00) << 8)
        p23 = ((lax.shift_right_logical(vx, 16) & 0xFF)
               | (lax.shift_right_logical(vx, 8) & 0xFF0000))
        s = sdup0 if w < 8 else sdup1
        f01 = (plsc.bitcast(p01, jnp.int16).astype(jnp.float16) - c128) * s
        f23 = (plsc.bitcast(p23, jnp.int16).astype(jnp.float16) - c128) * s
        plsc.store_scatter(out_v, [row_ids, jnp.full((16,), 2 * w, jnp.int32)],
                           plsc.bitcast(f01, jnp.int32))
        plsc.store_scatter(out_v, [row_ids, jnp.full((16,), 2 * w + 1, jnp.int32)],
                           plsc.bitcast(f23, jnp.int32))


def _sc_body(idx_hbm, w_hbm, s_hbm, out_hbm, idx_v, scl_v, rows_v, out_v,
             sem_w, sem_s):
    wid = lax.axis_index("s") * 2 + lax.axis_index("c")
    base_row = wid * _PER_W

    def chunk_body(c, _):
        r0 = pl.multiple_of(base_row + c * _CHUNK, _CHUNK)
        pltpu.sync_copy(idx_hbm.at[pl.ds(pl.multiple_of(r0 // _SUB, _NSUB), _NSUB)],
                        idx_v)
        copies = []
        for j in range(_NSUB):
            copies.append(pltpu.async_copy(
                w_hbm.at[idx_v.at[j]], rows_v.at[pl.ds(j * _SUB, _SUB)], sem_w))
            copies.append(pltpu.async_copy(
                s_hbm.at[idx_v.at[j]], scl_v.at[pl.ds(j * _SUB, _SUB)], sem_s))
        for cp in copies:
            cp.wait()

        def group_body(g, _):
            _dequant_group(rows_v, scl_v, out_v, g)
            return 0

        lax.fori_loop(0, _GROUPS, group_body, 0)
        pltpu.sync_copy(out_v, out_hbm.at[pl.ds(r0, _CHUNK)])
        return 0

    lax.fori_loop(0, _NCHUNK, chunk_body, 0)


@functools.partial(
    pl.kernel,
    out_type=jax.ShapeDtypeStruct((_N, 16 * 2), jnp.int32),
    mesh=plsc.VectorSubcoreMesh(core_axis_name="c", subcore_axis_name="s"),
    scratch_types=[
        pltpu.VMEM((_NSUB, _SUB), jnp.int32),
        pltpu.VMEM((_CHUNK,), jnp.int32),
        pltpu.VMEM((_CHUNK, 16), jnp.int32),
        pltpu.VMEM((_CHUNK, 32), jnp.int32),
        pltpu.SemaphoreType.DMA,
        pltpu.SemaphoreType.DMA,
    ],
    compiler_params=pltpu.CompilerParams(needs_layout_passes=False,
                                         use_tc_tiling_on_sc=False),
)
def _sc_lookup(idx_hbm, w_hbm, s_hbm, out_hbm, idx_v, scl_v, rows_v, out_v,
               sem_w, sem_s):
    _sc_body(idx_hbm, w_hbm, s_hbm, out_hbm, idx_v, scl_v, rows_v, out_v,
             sem_w, sem_s)


def kernel(indices, weight_int8, scales_fp16):
    V, D = weight_int8.shape
    idx2d = indices.reshape(_N // _SUB, _SUB)
    w_i32 = lax.bitcast_convert_type(weight_int8.reshape(V, D // 4, 4), jnp.int32)
    s_i32 = lax.bitcast_convert_type(scales_fp16, jnp.int32)
    out_i32 = _sc_lookup(idx2d, w_i32, s_i32)
    out = lax.bitcast_convert_type(out_i32, jnp.float16)
    return out.reshape(indices.shape + (D,))
